# Initial kernel scaffold; baseline (speedup 1.0000x reference)
import jax
import jax.numpy as jnp
from jax.experimental import pallas as pl


def kernel(pred_sid, pred_corr_factor, rechit_energy, no_noise_idx, pred_beta, is_track, alpha_idx_tracks, alpha_idx_hits):
    raise NotImplementedError("write your pallas kernel here")


# speedup vs baseline: 61.4055x; 61.4055x over previous
import jax
import jax.numpy as jnp
from jax import lax
from jax.experimental import pallas as pl
from jax.experimental.pallas import tpu as pltpu
from jax.experimental.pallas import tpu_sc as plsc

N = 1_600_000
S = 100_000
NC = 2
NS = 16
NW = NC * NS
LW = 128

S_PAD = 102_400
TBL = 2 * S_PAD
ZSL = TBL // NS
SSL = S_PAD // NS
SSL_ROWS = SSL // LW
CH_ROWS = 20
CH = CH_ROWS * LW
NCHUNKS = N // CH

_mesh = plsc.VectorSubcoreMesh(
    core_axis_name="c", subcore_axis_name="s", num_cores=NC, num_subcores=NS
)


def _worker_id():
    return lax.axis_index("s") * NC + lax.axis_index("c")


def _num_chunks_for(w):
    return (NCHUNKS - w + NW - 1) // NW


def _seg_sums_body(sid_hbm, trk_hbm, en_hbm, part_hbm,
                   tbl, sidb, trkb, enb, idxb, zbuf):
    c = lax.axis_index("c")
    s = lax.axis_index("s")
    w = _worker_id()

    def zvec(i, carry):
        zbuf[pl.ds(i * 16, 16)] = jnp.zeros((16,), jnp.float32)
        return carry

    lax.fori_loop(0, ZSL // 16, zvec, 0)
    pltpu.sync_copy(zbuf, tbl.at[pl.ds(s * ZSL, ZSL)])
    plsc.subcore_barrier()

    def chunk(i, carry):
        eb = (w + i * NW) * CH
        pltpu.sync_copy(sid_hbm.at[pl.ds(eb, CH)], sidb)
        pltpu.sync_copy(trk_hbm.at[pl.ds(eb, CH)], trkb)
        pltpu.sync_copy(en_hbm.at[pl.ds(eb, CH)], enb)

        def row(r, carry2):
            def grp(k, carry3):
                col = k * 16
                o = r * LW + col
                sv = sidb[pl.ds(o, 16)]
                tv = trkb[pl.ds(o, 16)]
                idxb[r, pl.ds(col, 16)] = sv + tv * S_PAD
                return carry3

            lax.fori_loop(0, LW // 16, grp, 0)
            pltpu.sync_copy(enb.at[pl.ds(r * LW, LW)], tbl.at[idxb.at[r]], add=True)
            return carry2

        lax.fori_loop(0, CH_ROWS, row, 0)
        return carry

    lax.fori_loop(0, _num_chunks_for(w), chunk, 0)
    plsc.subcore_barrier()
    pltpu.sync_copy(tbl.at[pl.ds(s * ZSL, ZSL)],
                    part_hbm.at[pl.ds(c * TBL + s * ZSL, ZSL)])


_seg_sums = pl.kernel(
    _seg_sums_body,
    out_type=jax.ShapeDtypeStruct((NC * TBL,), jnp.float32),
    mesh=_mesh,
    scratch_types=[
        pltpu.VMEM_SHARED((TBL,), jnp.float32),
        pltpu.VMEM((CH,), jnp.int32),
        pltpu.VMEM((CH,), jnp.int32),
        pltpu.VMEM((CH,), jnp.float32),
        pltpu.VMEM((CH_ROWS, LW), jnp.int32),
        pltpu.VMEM((ZSL,), jnp.float32),
    ],
)


def _tables_gather_body(part_hbm, corr_hbm, aih_hbm, ait_hbm, sid_hbm,
                        otraw, otcor, ohraw, ohcor,
                        t_hraw, t_hcor, t_traw, t_tcor,
                        pa, pb, ia, cg, vraw, vcor, sidb, ob):
    s = lax.axis_index("s")
    w = _worker_id()
    base = s * SSL

    for gi, (t_raw, t_cor, alpha_hbm) in enumerate((
            (t_hraw, t_hcor, aih_hbm), (t_traw, t_tcor, ait_hbm))):
        off = gi * S_PAD
        pltpu.sync_copy(part_hbm.at[pl.ds(off + base, SSL)], pa)
        pltpu.sync_copy(part_hbm.at[pl.ds(TBL + off + base, SSL)], pb)
        pltpu.sync_copy(alpha_hbm.at[pl.ds(base, SSL)], cg)

        def cvt(k, carry):
            o = k * 16
            ia[pl.ds(o, 16)] = cg[pl.ds(o, 16)].astype(jnp.int32)
            return carry

        lax.fori_loop(0, SSL // 16, cvt, 0)

        def gat_row(r, carry):
            o = r * LW
            pltpu.sync_copy(corr_hbm.at[ia.at[pl.ds(o, LW)]], cg.at[pl.ds(o, LW)])
            return carry

        lax.fori_loop(0, SSL_ROWS, gat_row, 0)

        def comb(k, carry):
            o = k * 16
            v = pa[pl.ds(o, 16)] + pb[pl.ds(o, 16)]
            vraw[pl.ds(o, 16)] = v
            vcor[pl.ds(o, 16)] = v * cg[pl.ds(o, 16)]
            return carry

        lax.fori_loop(0, SSL // 16, comb, 0)
        pltpu.sync_copy(vraw, t_raw.at[pl.ds(base, SSL)])
        pltpu.sync_copy(vcor, t_cor.at[pl.ds(base, SSL)])

    plsc.subcore_barrier()

    def chunk(i, carry):
        eb = (w + i * NW) * CH
        pltpu.sync_copy(sid_hbm.at[pl.ds(eb, CH)], sidb)
        for tref, oref in ((t_traw, otraw), (t_tcor, otcor),
                           (t_hraw, ohraw), (t_hcor, ohcor)):
            def g_row(r, carry2, tref=tref):
                o = r * LW
                pltpu.sync_copy(tref.at[sidb.at[pl.ds(o, LW)]], ob.at[pl.ds(o, LW)])
                return carry2

            lax.fori_loop(0, CH_ROWS, g_row, 0)
            pltpu.sync_copy(ob, oref.at[pl.ds(eb, CH)])
        return carry

    lax.fori_loop(0, _num_chunks_for(w), chunk, 0)


_tables_gather = pl.kernel(
    _tables_gather_body,
    out_type=[jax.ShapeDtypeStruct((N,), jnp.float32)] * 4,
    mesh=_mesh,
    scratch_types=[
        pltpu.VMEM_SHARED((S_PAD,), jnp.float32),
        pltpu.VMEM_SHARED((S_PAD,), jnp.float32),
        pltpu.VMEM_SHARED((S_PAD,), jnp.float32),
        pltpu.VMEM_SHARED((S_PAD,), jnp.float32),
        pltpu.VMEM((SSL,), jnp.float32),
        pltpu.VMEM((SSL,), jnp.float32),
        pltpu.VMEM((SSL,), jnp.int32),
        pltpu.VMEM((SSL,), jnp.float32),
        pltpu.VMEM((SSL,), jnp.float32),
        pltpu.VMEM((SSL,), jnp.float32),
        pltpu.VMEM((CH,), jnp.int32),
        pltpu.VMEM((CH,), jnp.float32),
    ],
)


def kernel(pred_sid, pred_corr_factor, rechit_energy, no_noise_idx,
           pred_beta, is_track, alpha_idx_tracks, alpha_idx_hits):
    del no_noise_idx, pred_beta
    sid1d = pred_sid.reshape(N)
    trk1d = is_track.astype(jnp.int32).reshape(N)
    en1d = rechit_energy.reshape(N)
    corr1d = pred_corr_factor.reshape(N)
    zpad = jnp.zeros((S_PAD - S,), jnp.float32)
    aih1d = jnp.concatenate([alpha_idx_hits.astype(jnp.float32), zpad])
    ait1d = jnp.concatenate([alpha_idx_tracks.astype(jnp.float32), zpad])

    part = _seg_sums(sid1d, trk1d, en1d)
    otraw, otcor, ohraw, ohcor = _tables_gather(part, corr1d, aih1d, ait1d, sid1d)

    def rs(x):
        return x.reshape(N, 1)

    return (rs(otraw), rs(otcor), rs(ohraw), rs(ohcor))


# speedup vs baseline: 85.4711x; 1.3919x over previous
import jax
import jax.numpy as jnp
from jax import lax
from jax.experimental import pallas as pl
from jax.experimental.pallas import tpu as pltpu
from jax.experimental.pallas import tpu_sc as plsc

N = 1_600_000
S = 100_000
NC = 2
NS = 16
NW = NC * NS
LW = 128

S_PAD = 102_400
TBL = 2 * S_PAD
ZSL = TBL // NS
SSL = S_PAD // NS
SSL_ROWS = SSL // LW
CH_ROWS = 20
CH = CH_ROWS * LW
NCHUNKS = N // CH

_mesh = plsc.VectorSubcoreMesh(
    core_axis_name="c", subcore_axis_name="s", num_cores=NC, num_subcores=NS
)


def _worker_id():
    return lax.axis_index("s") * NC + lax.axis_index("c")


def _num_chunks_for(w):
    return (NCHUNKS - w + NW - 1) // NW


def _seg_sums_body(sid_hbm, trk_hbm, en_hbm, part_hbm,
                   tbl, sidb, trkb, enb, idxb, zbuf, sem):
    c = lax.axis_index("c")
    s = lax.axis_index("s")
    w = _worker_id()

    def zvec(i, carry):
        zbuf[pl.ds(i * 16, 16)] = jnp.zeros((16,), jnp.float32)
        return carry

    lax.fori_loop(0, ZSL // 16, zvec, 0)
    pltpu.sync_copy(zbuf, tbl.at[pl.ds(s * ZSL, ZSL)])
    plsc.subcore_barrier()

    def chunk(i, carry):
        eb = (w + i * NW) * CH
        pltpu.sync_copy(sid_hbm.at[pl.ds(eb, CH)], sidb)
        pltpu.sync_copy(trk_hbm.at[pl.ds(eb, CH)], trkb)
        pltpu.sync_copy(en_hbm.at[pl.ds(eb, CH)], enb)

        def row(r, carry2):
            for k in range(LW // 16):
                col = k * 16
                o = r * LW + col
                sv = sidb[pl.ds(o, 16)]
                tv = trkb[pl.ds(o, 16)]
                idxb[r, pl.ds(col, 16)] = sv + tv * S_PAD
            return carry2

        lax.fori_loop(0, CH_ROWS, row, 0)
        descs = [
            pltpu.async_copy(enb.at[pl.ds(r * LW, LW)], tbl.at[idxb.at[r]],
                             sem, add=True)
            for r in range(CH_ROWS)
        ]
        for d in descs:
            d.wait()
        return carry

    lax.fori_loop(0, _num_chunks_for(w), chunk, 0)
    plsc.subcore_barrier()
    pltpu.sync_copy(tbl.at[pl.ds(s * ZSL, ZSL)],
                    part_hbm.at[pl.ds(c * TBL + s * ZSL, ZSL)])


_seg_sums = pl.kernel(
    _seg_sums_body,
    out_type=jax.ShapeDtypeStruct((NC * TBL,), jnp.float32),
    mesh=_mesh,
    scratch_types=[
        pltpu.VMEM_SHARED((TBL,), jnp.float32),
        pltpu.VMEM((CH,), jnp.int32),
        pltpu.VMEM((CH,), jnp.int32),
        pltpu.VMEM((CH,), jnp.float32),
        pltpu.VMEM((CH_ROWS, LW), jnp.int32),
        pltpu.VMEM((ZSL,), jnp.float32),
        pltpu.SemaphoreType.DMA,
    ],
)


def _tables_gather_body(part_hbm, corr_hbm, aih_hbm, ait_hbm, sid_hbm,
                        otraw, otcor, ohraw, ohcor,
                        t_hraw, t_hcor, t_traw, t_tcor,
                        pa, pb, ia, cg, vraw, vcor, sidb,
                        ob0, ob1, ob2, ob3, sem):
    s = lax.axis_index("s")
    w = _worker_id()
    base = s * SSL

    for gi, (t_raw, t_cor, alpha_hbm) in enumerate((
            (t_hraw, t_hcor, aih_hbm), (t_traw, t_tcor, ait_hbm))):
        off = gi * S_PAD
        pltpu.sync_copy(part_hbm.at[pl.ds(off + base, SSL)], pa)
        pltpu.sync_copy(part_hbm.at[pl.ds(TBL + off + base, SSL)], pb)
        pltpu.sync_copy(alpha_hbm.at[pl.ds(base, SSL)], cg)

        def cvt(k, carry):
            o = k * 16
            ia[pl.ds(o, 16)] = cg[pl.ds(o, 16)].astype(jnp.int32)
            return carry

        lax.fori_loop(0, SSL // 16, cvt, 0)

        descs = [
            pltpu.async_copy(corr_hbm.at[ia.at[pl.ds(r * LW, LW)]],
                             cg.at[pl.ds(r * LW, LW)], sem)
            for r in range(SSL_ROWS)
        ]
        for d in descs:
            d.wait()

        def comb(k, carry):
            o = k * 16
            v = pa[pl.ds(o, 16)] + pb[pl.ds(o, 16)]
            vraw[pl.ds(o, 16)] = v
            vcor[pl.ds(o, 16)] = v * cg[pl.ds(o, 16)]
            return carry

        lax.fori_loop(0, SSL // 16, comb, 0)
        pltpu.sync_copy(vraw, t_raw.at[pl.ds(base, SSL)])
        pltpu.sync_copy(vcor, t_cor.at[pl.ds(base, SSL)])

    plsc.subcore_barrier()

    def chunk(i, carry):
        eb = (w + i * NW) * CH
        pltpu.sync_copy(sid_hbm.at[pl.ds(eb, CH)], sidb)
        descs = [
            pltpu.async_copy(tref.at[sidb.at[pl.ds(r * LW, LW)]],
                             ob.at[pl.ds(r * LW, LW)], sem)
            for tref, ob in ((t_traw, ob0), (t_tcor, ob1),
                             (t_hraw, ob2), (t_hcor, ob3))
            for r in range(CH_ROWS)
        ]
        for d in descs:
            d.wait()
        for ob, oref in ((ob0, otraw), (ob1, otcor), (ob2, ohraw), (ob3, ohcor)):
            pltpu.sync_copy(ob, oref.at[pl.ds(eb, CH)])
        return carry

    lax.fori_loop(0, _num_chunks_for(w), chunk, 0)


_tables_gather = pl.kernel(
    _tables_gather_body,
    out_type=[jax.ShapeDtypeStruct((N,), jnp.float32)] * 4,
    mesh=_mesh,
    scratch_types=[
        pltpu.VMEM_SHARED((S_PAD,), jnp.float32),
        pltpu.VMEM_SHARED((S_PAD,), jnp.float32),
        pltpu.VMEM_SHARED((S_PAD,), jnp.float32),
        pltpu.VMEM_SHARED((S_PAD,), jnp.float32),
        pltpu.VMEM((SSL,), jnp.float32),
        pltpu.VMEM((SSL,), jnp.float32),
        pltpu.VMEM((SSL,), jnp.int32),
        pltpu.VMEM((SSL,), jnp.float32),
        pltpu.VMEM((SSL,), jnp.float32),
        pltpu.VMEM((SSL,), jnp.float32),
        pltpu.VMEM((CH,), jnp.int32),
        pltpu.VMEM((CH,), jnp.float32),
        pltpu.VMEM((CH,), jnp.float32),
        pltpu.VMEM((CH,), jnp.float32),
        pltpu.VMEM((CH,), jnp.float32),
        pltpu.SemaphoreType.DMA,
    ],
)


def kernel(pred_sid, pred_corr_factor, rechit_energy, no_noise_idx,
           pred_beta, is_track, alpha_idx_tracks, alpha_idx_hits):
    del no_noise_idx, pred_beta
    sid1d = pred_sid.reshape(N)
    trk1d = is_track.astype(jnp.int32).reshape(N)
    en1d = rechit_energy.reshape(N)
    corr1d = pred_corr_factor.reshape(N)
    zpad = jnp.zeros((S_PAD - S,), jnp.float32)
    aih1d = jnp.concatenate([alpha_idx_hits.astype(jnp.float32), zpad])
    ait1d = jnp.concatenate([alpha_idx_tracks.astype(jnp.float32), zpad])

    part = _seg_sums(sid1d, trk1d, en1d)
    otraw, otcor, ohraw, ohcor = _tables_gather(part, corr1d, aih1d, ait1d, sid1d)

    def rs(x):
        return x.reshape(N, 1)

    return (rs(otraw), rs(otcor), rs(ohraw), rs(ohcor))


# speedup vs baseline: 126.6380x; 1.4816x over previous
import jax
import jax.numpy as jnp
from jax import lax
from jax.experimental import pallas as pl
from jax.experimental.pallas import tpu as pltpu
from jax.experimental.pallas import tpu_sc as plsc

N = 1_600_000
NP = 1_600_512
S = 100_000
NC = 2
NS = 16
NW = NC * NS

S_PAD = 102_400
TBL = 2 * S_PAD
ZSL = TBL // NS
SSL = S_PAD // NS
CH = 3072
NCHUNKS = NP // CH

_mesh = plsc.VectorSubcoreMesh(
    core_axis_name="c", subcore_axis_name="s", num_cores=NC, num_subcores=NS
)


def _worker_id():
    return lax.axis_index("s") * NC + lax.axis_index("c")


def _num_chunks_for(w):
    return (NCHUNKS - w + NW - 1) // NW


def _seg_sums_body(sid_hbm, trk_hbm, en_hbm, part_hbm,
                   tbl, sidb, trkb, enb, idxb, zbuf, sem):
    c = lax.axis_index("c")
    s = lax.axis_index("s")
    w = _worker_id()

    def zvec(i, carry):
        zbuf[pl.ds(i * 16, 16)] = jnp.zeros((16,), jnp.float32)
        return carry

    lax.fori_loop(0, ZSL // 16, zvec, 0)
    pltpu.sync_copy(zbuf, tbl.at[pl.ds(s * ZSL, ZSL)])
    plsc.subcore_barrier()

    def chunk(i, carry):
        eb = (w + i * NW) * CH
        pltpu.sync_copy(sid_hbm.at[pl.ds(eb, CH)], sidb)
        pltpu.sync_copy(trk_hbm.at[pl.ds(eb, CH)], trkb)
        pltpu.sync_copy(en_hbm.at[pl.ds(eb, CH)], enb)

        def grp(j, carry2):
            for u in range(8):
                o = j * 128 + u * 16
                idxb[pl.ds(o, 16)] = sidb[pl.ds(o, 16)] + trkb[pl.ds(o, 16)] * S_PAD
            return carry2

        lax.fori_loop(0, CH // 128, grp, 0)
        pltpu.async_copy(enb, tbl.at[idxb], sem, add=True).wait()
        return carry

    lax.fori_loop(0, _num_chunks_for(w), chunk, 0)
    plsc.subcore_barrier()
    pltpu.sync_copy(tbl.at[pl.ds(s * ZSL, ZSL)],
                    part_hbm.at[pl.ds(c * TBL + s * ZSL, ZSL)])


_seg_sums = pl.kernel(
    _seg_sums_body,
    out_type=jax.ShapeDtypeStruct((NC * TBL,), jnp.float32),
    mesh=_mesh,
    scratch_types=[
        pltpu.VMEM_SHARED((TBL,), jnp.float32),
        pltpu.VMEM((CH,), jnp.int32),
        pltpu.VMEM((CH,), jnp.int32),
        pltpu.VMEM((CH,), jnp.float32),
        pltpu.VMEM((CH,), jnp.int32),
        pltpu.VMEM((ZSL,), jnp.float32),
        pltpu.SemaphoreType.DMA,
    ],
)


def _tables_gather_body(part_hbm, corr_hbm, aih_hbm, ait_hbm, sid_hbm,
                        otraw, otcor, ohraw, ohcor,
                        t_hraw, t_hcor, t_traw, t_tcor,
                        pa, pb, ia, cg, vraw, vcor, sidb,
                        ob0, ob1, ob2, ob3, sem):
    s = lax.axis_index("s")
    w = _worker_id()
    base = s * SSL

    for gi, (t_raw, t_cor, alpha_hbm) in enumerate((
            (t_hraw, t_hcor, aih_hbm), (t_traw, t_tcor, ait_hbm))):
        off = gi * S_PAD
        pltpu.sync_copy(part_hbm.at[pl.ds(off + base, SSL)], pa)
        pltpu.sync_copy(part_hbm.at[pl.ds(TBL + off + base, SSL)], pb)
        pltpu.sync_copy(alpha_hbm.at[pl.ds(base, SSL)], cg)

        def cvt(k, carry):
            o = k * 16
            ia[pl.ds(o, 16)] = cg[pl.ds(o, 16)].astype(jnp.int32)
            return carry

        lax.fori_loop(0, SSL // 16, cvt, 0)
        pltpu.async_copy(corr_hbm.at[ia], cg, sem).wait()

        def comb(k, carry):
            o = k * 16
            v = pa[pl.ds(o, 16)] + pb[pl.ds(o, 16)]
            vraw[pl.ds(o, 16)] = v
            vcor[pl.ds(o, 16)] = v * cg[pl.ds(o, 16)]
            return carry

        lax.fori_loop(0, SSL // 16, comb, 0)
        pltpu.sync_copy(vraw, t_raw.at[pl.ds(base, SSL)])
        pltpu.sync_copy(vcor, t_cor.at[pl.ds(base, SSL)])

    plsc.subcore_barrier()

    def chunk(i, carry):
        eb = (w + i * NW) * CH
        pltpu.sync_copy(sid_hbm.at[pl.ds(eb, CH)], sidb)
        descs = [
            pltpu.async_copy(tref.at[sidb], ob, sem)
            for tref, ob in ((t_traw, ob0), (t_tcor, ob1),
                             (t_hraw, ob2), (t_hcor, ob3))
        ]
        for d in descs:
            d.wait()
        for ob, oref in ((ob0, otraw), (ob1, otcor), (ob2, ohraw), (ob3, ohcor)):
            pltpu.sync_copy(ob, oref.at[pl.ds(eb, CH)])
        return carry

    lax.fori_loop(0, _num_chunks_for(w), chunk, 0)


_tables_gather = pl.kernel(
    _tables_gather_body,
    out_type=[jax.ShapeDtypeStruct((NP,), jnp.float32)] * 4,
    mesh=_mesh,
    scratch_types=[
        pltpu.VMEM_SHARED((S_PAD,), jnp.float32),
        pltpu.VMEM_SHARED((S_PAD,), jnp.float32),
        pltpu.VMEM_SHARED((S_PAD,), jnp.float32),
        pltpu.VMEM_SHARED((S_PAD,), jnp.float32),
        pltpu.VMEM((SSL,), jnp.float32),
        pltpu.VMEM((SSL,), jnp.float32),
        pltpu.VMEM((SSL,), jnp.int32),
        pltpu.VMEM((SSL,), jnp.float32),
        pltpu.VMEM((SSL,), jnp.float32),
        pltpu.VMEM((SSL,), jnp.float32),
        pltpu.VMEM((CH,), jnp.int32),
        pltpu.VMEM((CH,), jnp.float32),
        pltpu.VMEM((CH,), jnp.float32),
        pltpu.VMEM((CH,), jnp.float32),
        pltpu.VMEM((CH,), jnp.float32),
        pltpu.SemaphoreType.DMA,
    ],
)


def _squeeze_pad(x):
    return jnp.pad(x, ((0, NP - N), (0, 0))).reshape(NP)


def kernel(pred_sid, pred_corr_factor, rechit_energy, no_noise_idx,
           pred_beta, is_track, alpha_idx_tracks, alpha_idx_hits):
    del no_noise_idx, pred_beta
    sid1d = _squeeze_pad(pred_sid)
    trk1d = _squeeze_pad(is_track.astype(jnp.int32))
    en1d = _squeeze_pad(rechit_energy)
    corr1d = _squeeze_pad(pred_corr_factor)
    zpad = jnp.zeros((S_PAD - S,), jnp.float32)
    aih1d = jnp.concatenate([alpha_idx_hits.astype(jnp.float32), zpad])
    ait1d = jnp.concatenate([alpha_idx_tracks.astype(jnp.float32), zpad])

    part = _seg_sums(sid1d, trk1d, en1d)
    otraw, otcor, ohraw, ohcor = _tables_gather(part, corr1d, aih1d, ait1d, sid1d)

    def rs(x):
        return lax.slice(x.reshape(NP, 1), (0, 0), (N, 1))

    return (rs(otraw), rs(otcor), rs(ohraw), rs(ohcor))


# speedup vs baseline: 138.2486x; 1.0917x over previous
import jax
import jax.numpy as jnp
from jax import lax
from jax.experimental import pallas as pl
from jax.experimental.pallas import tpu as pltpu
from jax.experimental.pallas import tpu_sc as plsc

N = 1_600_000
NP = 1_600_512
S = 100_000
NC = 2
NS = 16
NW = NC * NS

S_PAD = 102_400
TBL = 2 * S_PAD
ZSL = TBL // NS
SSL = S_PAD // NS
CH = 3072
NCHUNKS = NP // CH

_mesh = plsc.VectorSubcoreMesh(
    core_axis_name="c", subcore_axis_name="s", num_cores=NC, num_subcores=NS
)


def _worker_id():
    return lax.axis_index("s") * NC + lax.axis_index("c")


def _num_chunks_for(w):
    return (NCHUNKS - w + NW - 1) // NW


def _seg_sums_body(sid_hbm, trk_hbm, en_hbm, part_hbm,
                   tbl, sidb, trkb, enb, idxb, zbuf, sem):
    c = lax.axis_index("c")
    s = lax.axis_index("s")
    w = _worker_id()

    def zvec(i, carry):
        zbuf[pl.ds(i * 16, 16)] = jnp.zeros((16,), jnp.float32)
        return carry

    lax.fori_loop(0, ZSL // 16, zvec, 0)
    pltpu.sync_copy(zbuf, tbl.at[pl.ds(s * ZSL, ZSL)])
    plsc.subcore_barrier()

    def chunk(i, carry):
        eb = (w + i * NW) * CH
        pltpu.sync_copy(sid_hbm.at[pl.ds(eb, CH)], sidb)
        pltpu.sync_copy(trk_hbm.at[pl.ds(eb, CH)], trkb)
        pltpu.sync_copy(en_hbm.at[pl.ds(eb, CH)], enb)

        def grp(j, carry2):
            for u in range(8):
                o = j * 128 + u * 16
                idxb[pl.ds(o, 16)] = sidb[pl.ds(o, 16)] + trkb[pl.ds(o, 16)] * S_PAD
            return carry2

        lax.fori_loop(0, CH // 128, grp, 0)
        pltpu.async_copy(enb, tbl.at[idxb], sem, add=True).wait()
        return carry

    lax.fori_loop(0, _num_chunks_for(w), chunk, 0)
    plsc.subcore_barrier()
    pltpu.sync_copy(tbl.at[pl.ds(s * ZSL, ZSL)],
                    part_hbm.at[pl.ds(c * TBL + s * ZSL, ZSL)])


_seg_sums = pl.kernel(
    _seg_sums_body,
    out_type=jax.ShapeDtypeStruct((NC * TBL,), jnp.float32),
    mesh=_mesh,
    scratch_types=[
        pltpu.VMEM_SHARED((TBL,), jnp.float32),
        pltpu.VMEM((CH,), jnp.int32),
        pltpu.VMEM((CH,), jnp.int32),
        pltpu.VMEM((CH,), jnp.float32),
        pltpu.VMEM((CH,), jnp.int32),
        pltpu.VMEM((ZSL,), jnp.float32),
        pltpu.SemaphoreType.DMA,
    ],
)


def _pair_gather_loop(sid_hbm, ta, tb, oa, ob, sidb, b0, b1, sem, w):
    def chunk(i, carry):
        eb = (w + i * NW) * CH
        pltpu.sync_copy(sid_hbm.at[pl.ds(eb, CH)], sidb)
        d1 = pltpu.async_copy(ta.at[sidb], b0, sem)
        d2 = pltpu.async_copy(tb.at[sidb], b1, sem)
        d1.wait()
        d2.wait()
        pltpu.sync_copy(b0, oa.at[pl.ds(eb, CH)])
        pltpu.sync_copy(b1, ob.at[pl.ds(eb, CH)])
        return carry

    lax.fori_loop(0, _num_chunks_for(w), chunk, 0)


def _track_body(part_hbm, corr_hbm, aih_hbm, ait_hbm, sid_hbm,
                tblh_hbm, otraw, otcor,
                t_traw, t_tcor, pa, pb, ia, cg, vraw, vcor,
                sidb, b0, b1, sem):
    s = lax.axis_index("s")
    w = _worker_id()
    base = s * SSL

    for gi, alpha_hbm in ((0, aih_hbm), (1, ait_hbm)):
        off = gi * S_PAD
        pltpu.sync_copy(part_hbm.at[pl.ds(off + base, SSL)], pa)
        pltpu.sync_copy(part_hbm.at[pl.ds(TBL + off + base, SSL)], pb)
        pltpu.sync_copy(alpha_hbm.at[pl.ds(base, SSL)], cg)

        def cvt(k, carry):
            o = k * 16
            ia[pl.ds(o, 16)] = cg[pl.ds(o, 16)].astype(jnp.int32)
            return carry

        lax.fori_loop(0, SSL // 16, cvt, 0)
        pltpu.async_copy(corr_hbm.at[ia], cg, sem).wait()

        def comb(k, carry):
            o = k * 16
            v = pa[pl.ds(o, 16)] + pb[pl.ds(o, 16)]
            vraw[pl.ds(o, 16)] = v
            vcor[pl.ds(o, 16)] = v * cg[pl.ds(o, 16)]
            return carry

        lax.fori_loop(0, SSL // 16, comb, 0)
        if gi == 0:
            @pl.when(lax.axis_index("c") == 0)
            def _dump():
                pltpu.sync_copy(vraw, tblh_hbm.at[pl.ds(base, SSL)])
                pltpu.sync_copy(vcor, tblh_hbm.at[pl.ds(S_PAD + base, SSL)])
        else:
            pltpu.sync_copy(vraw, t_traw.at[pl.ds(base, SSL)])
            pltpu.sync_copy(vcor, t_tcor.at[pl.ds(base, SSL)])

    plsc.subcore_barrier()
    _pair_gather_loop(sid_hbm, t_traw, t_tcor, otraw, otcor,
                      sidb, b0, b1, sem, w)


_track_kernel = pl.kernel(
    _track_body,
    out_type=[jax.ShapeDtypeStruct((2 * S_PAD,), jnp.float32),
              jax.ShapeDtypeStruct((NP,), jnp.float32),
              jax.ShapeDtypeStruct((NP,), jnp.float32)],
    mesh=_mesh,
    scratch_types=[
        pltpu.VMEM_SHARED((S_PAD,), jnp.float32),
        pltpu.VMEM_SHARED((S_PAD,), jnp.float32),
        pltpu.VMEM((SSL,), jnp.float32),
        pltpu.VMEM((SSL,), jnp.float32),
        pltpu.VMEM((SSL,), jnp.int32),
        pltpu.VMEM((SSL,), jnp.float32),
        pltpu.VMEM((SSL,), jnp.float32),
        pltpu.VMEM((SSL,), jnp.float32),
        pltpu.VMEM((CH,), jnp.int32),
        pltpu.VMEM((CH,), jnp.float32),
        pltpu.VMEM((CH,), jnp.float32),
        pltpu.SemaphoreType.DMA,
    ],
)


def _hit_body(tblh_hbm, sid_hbm, ohraw, ohcor,
              t_hraw, t_hcor, sidb, b0, b1, sem):
    s = lax.axis_index("s")
    w = _worker_id()
    base = s * SSL
    pltpu.sync_copy(tblh_hbm.at[pl.ds(base, SSL)], t_hraw.at[pl.ds(base, SSL)])
    pltpu.sync_copy(tblh_hbm.at[pl.ds(S_PAD + base, SSL)],
                    t_hcor.at[pl.ds(base, SSL)])
    plsc.subcore_barrier()
    _pair_gather_loop(sid_hbm, t_hraw, t_hcor, ohraw, ohcor,
                      sidb, b0, b1, sem, w)


_hit_kernel = pl.kernel(
    _hit_body,
    out_type=[jax.ShapeDtypeStruct((NP,), jnp.float32)] * 2,
    mesh=_mesh,
    scratch_types=[
        pltpu.VMEM_SHARED((S_PAD,), jnp.float32),
        pltpu.VMEM_SHARED((S_PAD,), jnp.float32),
        pltpu.VMEM((CH,), jnp.int32),
        pltpu.VMEM((CH,), jnp.float32),
        pltpu.VMEM((CH,), jnp.float32),
        pltpu.SemaphoreType.DMA,
    ],
)


def _squeeze_pad(x):
    return jnp.pad(x, ((0, NP - N), (0, 0))).reshape(NP)


def kernel(pred_sid, pred_corr_factor, rechit_energy, no_noise_idx,
           pred_beta, is_track, alpha_idx_tracks, alpha_idx_hits):
    del no_noise_idx, pred_beta
    sid1d = _squeeze_pad(pred_sid)
    trk1d = _squeeze_pad(is_track.astype(jnp.int32))
    en1d = _squeeze_pad(rechit_energy)
    corr1d = _squeeze_pad(pred_corr_factor)
    zpad = jnp.zeros((S_PAD - S,), jnp.float32)
    aih1d = jnp.concatenate([alpha_idx_hits.astype(jnp.float32), zpad])
    ait1d = jnp.concatenate([alpha_idx_tracks.astype(jnp.float32), zpad])

    part = _seg_sums(sid1d, trk1d, en1d)
    tblh, otraw, otcor = _track_kernel(part, corr1d, aih1d, ait1d, sid1d)
    ohraw, ohcor = _hit_kernel(tblh, sid1d)

    def rs(x):
        return lax.slice(x.reshape(NP, 1), (0, 0), (N, 1))

    return (rs(otraw), rs(otcor), rs(ohraw), rs(ohcor))


# speedup vs baseline: 143.0364x; 1.0346x over previous
import jax
import jax.numpy as jnp
from jax import lax
from jax.experimental import pallas as pl
from jax.experimental.pallas import tpu as pltpu
from jax.experimental.pallas import tpu_sc as plsc

N = 1_600_000
NP = 1_600_512
S = 100_000
NC = 2
NS = 16
NW = NC * NS

S_PAD = 102_400
TBL = 2 * S_PAD
ZSL = TBL // NS
SSL = S_PAD // NS
CH = 3072
NCHUNKS = NP // CH

_mesh = plsc.VectorSubcoreMesh(
    core_axis_name="c", subcore_axis_name="s", num_cores=NC, num_subcores=NS
)


def _worker_id():
    return lax.axis_index("s") * NC + lax.axis_index("c")


def _num_chunks_for(w):
    return (NCHUNKS - w + NW - 1) // NW


HC = CH // 2


def _seg_sums_body(sid_hbm, trk_hbm, en_hbm, part_hbm,
                   tbl, sidb, trkb, enbA, enbB, idxa, idxb2, zbuf, semi, sems):
    c = lax.axis_index("c")
    s = lax.axis_index("s")
    w = _worker_id()
    n = _num_chunks_for(w)

    def zvec(i, carry):
        zbuf[pl.ds(i * 16, 16)] = jnp.zeros((16,), jnp.float32)
        return carry

    lax.fori_loop(0, ZSL // 16, zvec, 0)
    pltpu.sync_copy(zbuf, tbl.at[pl.ds(s * ZSL, ZSL)])
    plsc.subcore_barrier()

    pltpu.sync_copy(sid_hbm.at[pl.ds(w * CH, CH)], sidb)
    pltpu.sync_copy(trk_hbm.at[pl.ds(w * CH, CH)], trkb)

    def chunk(i, carry):
        eb = (w + i * NW) * CH

        @pl.when(i > 0)
        def _drain_prefetch():
            pltpu.make_async_copy(sid_hbm.at[pl.ds(eb, CH)], sidb, semi).wait()
            pltpu.make_async_copy(trk_hbm.at[pl.ds(eb, CH)], trkb, semi).wait()

        dl1 = pltpu.async_copy(en_hbm.at[pl.ds(eb, HC)], enbA, semi)
        dl2 = pltpu.async_copy(en_hbm.at[pl.ds(eb + HC, HC)], enbB, semi)

        def mk_grp(idx_ref, half):
            def grp(j, carry2):
                for u in range(8):
                    o = half * HC + j * 128 + u * 16
                    idx_ref[pl.ds(j * 128 + u * 16, 16)] = (
                        sidb[pl.ds(o, 16)] + trkb[pl.ds(o, 16)] * S_PAD)
                return carry2
            return grp

        lax.fori_loop(0, HC // 128, mk_grp(idxa, 0), 0)
        dl1.wait()
        dl2.wait()
        dA = pltpu.async_copy(enbA, tbl.at[idxa], sems, add=True)
        lax.fori_loop(0, HC // 128, mk_grp(idxb2, 1), 0)
        dB = pltpu.async_copy(enbB, tbl.at[idxb2], sems, add=True)

        @pl.when(i + 1 < n)
        def _prefetch():
            eb2 = (w + (i + 1) * NW) * CH
            pltpu.async_copy(sid_hbm.at[pl.ds(eb2, CH)], sidb, semi)
            pltpu.async_copy(trk_hbm.at[pl.ds(eb2, CH)], trkb, semi)

        dA.wait()
        dB.wait()
        return carry

    lax.fori_loop(0, n, chunk, 0)
    plsc.subcore_barrier()
    pltpu.sync_copy(tbl.at[pl.ds(s * ZSL, ZSL)],
                    part_hbm.at[pl.ds(c * TBL + s * ZSL, ZSL)])


_seg_sums = pl.kernel(
    _seg_sums_body,
    out_type=jax.ShapeDtypeStruct((NC * TBL,), jnp.float32),
    mesh=_mesh,
    scratch_types=[
        pltpu.VMEM_SHARED((TBL,), jnp.float32),
        pltpu.VMEM((CH,), jnp.int32),
        pltpu.VMEM((CH,), jnp.int32),
        pltpu.VMEM((HC,), jnp.float32),
        pltpu.VMEM((HC,), jnp.float32),
        pltpu.VMEM((HC,), jnp.int32),
        pltpu.VMEM((HC,), jnp.int32),
        pltpu.VMEM((ZSL,), jnp.float32),
        pltpu.SemaphoreType.DMA,
        pltpu.SemaphoreType.DMA,
    ],
)


def _pair_gather_loop(sid_hbm, ta, tb, oa, ob,
                      sidba, sidbb, a0, a1, b0, b1, semg, semw, sems, w):
    n = _num_chunks_for(w)

    def chunk(i, carry):
        eb = (w + i * NW) * CH
        pltpu.sync_copy(sid_hbm.at[pl.ds(eb, HC)], sidba)
        pltpu.sync_copy(sid_hbm.at[pl.ds(eb + HC, HC)], sidbb)
        gA1 = pltpu.async_copy(ta.at[sidba], a0, semg)
        gA2 = pltpu.async_copy(tb.at[sidba], a1, semg)
        gB1 = pltpu.async_copy(ta.at[sidbb], b0, semg)
        gB2 = pltpu.async_copy(tb.at[sidbb], b1, semg)
        gA1.wait()
        gA2.wait()
        gB1.wait()
        gB2.wait()
        pltpu.sync_copy(a0, oa.at[pl.ds(eb, HC)])
        pltpu.sync_copy(a1, ob.at[pl.ds(eb, HC)])
        pltpu.sync_copy(b0, oa.at[pl.ds(eb + HC, HC)])
        pltpu.sync_copy(b1, ob.at[pl.ds(eb + HC, HC)])
        return carry

    lax.fori_loop(0, n, chunk, 0)


def _track_body(part_hbm, corr_hbm, aih_hbm, ait_hbm, sid_hbm,
                tblh_hbm, otraw, otcor,
                t_traw, t_tcor, pa0, pb0, pa1, pb1, ia0, ia1, cg0, cg1,
                vraw0, vcor0, vraw1, vcor1,
                sidba, sidbb, a0, a1, b0, b1,
                semi, semg, semw, sems):
    s = lax.axis_index("s")
    c = lax.axis_index("c")
    w = _worker_id()
    base = s * SSL

    g = ((pa0, pb0, ia0, cg0, vraw0, vcor0, aih_hbm),
         (pa1, pb1, ia1, cg1, vraw1, vcor1, ait_hbm))
    loads = []
    for gi, (pa, pb, ia, cg, vraw, vcor, alpha_hbm) in enumerate(g):
        off = gi * S_PAD
        loads.append((
            pltpu.async_copy(part_hbm.at[pl.ds(off + base, SSL)], pa, semi),
            pltpu.async_copy(part_hbm.at[pl.ds(TBL + off + base, SSL)], pb, semi),
            pltpu.async_copy(alpha_hbm.at[pl.ds(base, SSL)], cg, semi),
        ))
    for dtrip in loads:
        for d in dtrip:
            d.wait()
    gath = []
    for gi, (pa, pb, ia, cg, vraw, vcor, alpha_hbm) in enumerate(g):

        def cvt(k, carry, ia=ia, cg=cg):
            o = k * 16
            ia[pl.ds(o, 16)] = cg[pl.ds(o, 16)].astype(jnp.int32)
            return carry

        lax.fori_loop(0, SSL // 16, cvt, 0)
        gath.append(pltpu.async_copy(corr_hbm.at[ia], cg, semg))
    gath[0].wait()
    gath[1].wait()
    for gi, (pa, pb, ia, cg, vraw, vcor, alpha_hbm) in enumerate(g):

        def comb(k, carry, pa=pa, pb=pb, cg=cg, vraw=vraw, vcor=vcor):
            o = k * 16
            v = pa[pl.ds(o, 16)] + pb[pl.ds(o, 16)]
            vraw[pl.ds(o, 16)] = v
            vcor[pl.ds(o, 16)] = v * cg[pl.ds(o, 16)]
            return carry

        lax.fori_loop(0, SSL // 16, comb, 0)
        if gi == 0:
            @pl.when(c == 0)
            def _dump():
                pltpu.async_copy(vraw0, tblh_hbm.at[pl.ds(base, SSL)], semw)
                pltpu.async_copy(vcor0, tblh_hbm.at[pl.ds(S_PAD + base, SSL)], semw)
        else:
            pltpu.sync_copy(vraw1, t_traw.at[pl.ds(base, SSL)])
            pltpu.sync_copy(vcor1, t_tcor.at[pl.ds(base, SSL)])

    @pl.when(c == 0)
    def _drain_dump():
        pltpu.make_async_copy(vraw0, tblh_hbm.at[pl.ds(base, SSL)], semw).wait()
        pltpu.make_async_copy(vcor0, tblh_hbm.at[pl.ds(S_PAD + base, SSL)], semw).wait()

    plsc.subcore_barrier()
    _pair_gather_loop(sid_hbm, t_traw, t_tcor, otraw, otcor,
                      sidba, sidbb, a0, a1, b0, b1, semg, semw, sems, w)


_track_kernel = pl.kernel(
    _track_body,
    out_type=[jax.ShapeDtypeStruct((2 * S_PAD,), jnp.float32),
              jax.ShapeDtypeStruct((NP,), jnp.float32),
              jax.ShapeDtypeStruct((NP,), jnp.float32)],
    mesh=_mesh,
    scratch_types=[
        pltpu.VMEM_SHARED((S_PAD,), jnp.float32),
        pltpu.VMEM_SHARED((S_PAD,), jnp.float32),
        pltpu.VMEM((SSL,), jnp.float32),
        pltpu.VMEM((SSL,), jnp.float32),
        pltpu.VMEM((SSL,), jnp.float32),
        pltpu.VMEM((SSL,), jnp.float32),
        pltpu.VMEM((SSL,), jnp.int32),
        pltpu.VMEM((SSL,), jnp.int32),
        pltpu.VMEM((SSL,), jnp.float32),
        pltpu.VMEM((SSL,), jnp.float32),
        pltpu.VMEM((SSL,), jnp.float32),
        pltpu.VMEM((SSL,), jnp.float32),
        pltpu.VMEM((SSL,), jnp.float32),
        pltpu.VMEM((SSL,), jnp.float32),
        pltpu.VMEM((HC,), jnp.int32),
        pltpu.VMEM((HC,), jnp.int32),
        pltpu.VMEM((HC,), jnp.float32),
        pltpu.VMEM((HC,), jnp.float32),
        pltpu.VMEM((HC,), jnp.float32),
        pltpu.VMEM((HC,), jnp.float32),
        pltpu.SemaphoreType.DMA,
        pltpu.SemaphoreType.DMA,
        pltpu.SemaphoreType.DMA,
        pltpu.SemaphoreType.DMA,
    ],
)


def _hit_body(tblh_hbm, sid_hbm, ohraw, ohcor,
              t_hraw, t_hcor, sidba, sidbb, a0, a1, b0, b1,
              semi, semg, semw, sems):
    s = lax.axis_index("s")
    w = _worker_id()
    base = s * SSL
    d1 = pltpu.async_copy(tblh_hbm.at[pl.ds(base, SSL)],
                          t_hraw.at[pl.ds(base, SSL)], semi)
    d2 = pltpu.async_copy(tblh_hbm.at[pl.ds(S_PAD + base, SSL)],
                          t_hcor.at[pl.ds(base, SSL)], semi)
    d1.wait()
    d2.wait()
    plsc.subcore_barrier()
    _pair_gather_loop(sid_hbm, t_hraw, t_hcor, ohraw, ohcor,
                      sidba, sidbb, a0, a1, b0, b1, semg, semw, sems, w)


_hit_kernel = pl.kernel(
    _hit_body,
    out_type=[jax.ShapeDtypeStruct((NP,), jnp.float32)] * 2,
    mesh=_mesh,
    scratch_types=[
        pltpu.VMEM_SHARED((S_PAD,), jnp.float32),
        pltpu.VMEM_SHARED((S_PAD,), jnp.float32),
        pltpu.VMEM((HC,), jnp.int32),
        pltpu.VMEM((HC,), jnp.int32),
        pltpu.VMEM((HC,), jnp.float32),
        pltpu.VMEM((HC,), jnp.float32),
        pltpu.VMEM((HC,), jnp.float32),
        pltpu.VMEM((HC,), jnp.float32),
        pltpu.SemaphoreType.DMA,
        pltpu.SemaphoreType.DMA,
        pltpu.SemaphoreType.DMA,
        pltpu.SemaphoreType.DMA,
    ],
)


def _squeeze_pad(x):
    return jnp.pad(x, ((0, NP - N), (0, 0))).reshape(NP)


def kernel(pred_sid, pred_corr_factor, rechit_energy, no_noise_idx,
           pred_beta, is_track, alpha_idx_tracks, alpha_idx_hits):
    del no_noise_idx, pred_beta
    sid1d = _squeeze_pad(pred_sid)
    trk1d = _squeeze_pad(is_track.astype(jnp.int32))
    en1d = _squeeze_pad(rechit_energy)
    corr1d = _squeeze_pad(pred_corr_factor)
    zpad = jnp.zeros((S_PAD - S,), jnp.float32)
    aih1d = jnp.concatenate([alpha_idx_hits.astype(jnp.float32), zpad])
    ait1d = jnp.concatenate([alpha_idx_tracks.astype(jnp.float32), zpad])

    part = _seg_sums(sid1d, trk1d, en1d)
    tblh, otraw, otcor = _track_kernel(part, corr1d, aih1d, ait1d, sid1d)
    ohraw, ohcor = _hit_kernel(tblh, sid1d)

    def rs(x):
        return lax.slice(x.reshape(NP, 1), (0, 0), (N, 1))

    return (rs(otraw), rs(otcor), rs(ohraw), rs(ohcor))


# speedup vs baseline: 163.1357x; 1.1405x over previous
import jax
import jax.numpy as jnp
from jax import lax
from jax.experimental import pallas as pl
from jax.experimental.pallas import tpu as pltpu
from jax.experimental.pallas import tpu_sc as plsc

N = 1_600_000
NP = 1_600_512
S = 100_000
NC = 2
NS = 16
NW = NC * NS

S_PAD = 102_400
TBL = 2 * S_PAD
ZSL = TBL // NS
SSL = S_PAD // NS
CH = 3072
NCHUNKS = NP // CH

_mesh = plsc.VectorSubcoreMesh(
    core_axis_name="c", subcore_axis_name="s", num_cores=NC, num_subcores=NS
)


def _worker_id():
    return lax.axis_index("s") * NC + lax.axis_index("c")


def _num_chunks_for(w):
    return (NCHUNKS - w + NW - 1) // NW


HC = CH // 2


def _seg_sums_body(sid_hbm, trk_hbm, en_hbm, part_hbm,
                   tbl, sidb, trkb, enbA, enbB, idxa, idxb2, zbuf, semi, sems):
    c = lax.axis_index("c")
    s = lax.axis_index("s")
    w = _worker_id()
    n = _num_chunks_for(w)

    def zvec(i, carry):
        zbuf[pl.ds(i * 16, 16)] = jnp.zeros((16,), jnp.float32)
        return carry

    lax.fori_loop(0, ZSL // 16, zvec, 0)
    pltpu.sync_copy(zbuf, tbl.at[pl.ds(s * ZSL, ZSL)])
    plsc.subcore_barrier()

    pltpu.sync_copy(sid_hbm.at[pl.ds(w * CH, CH)], sidb)
    pltpu.sync_copy(trk_hbm.at[pl.ds(w * CH, CH)], trkb)

    def chunk(i, carry):
        eb = (w + i * NW) * CH

        @pl.when(i > 0)
        def _drain_prefetch():
            pltpu.make_async_copy(sid_hbm.at[pl.ds(eb, CH)], sidb, semi).wait()
            pltpu.make_async_copy(trk_hbm.at[pl.ds(eb, CH)], trkb, semi).wait()

        dl1 = pltpu.async_copy(en_hbm.at[pl.ds(eb, HC)], enbA, semi)
        dl2 = pltpu.async_copy(en_hbm.at[pl.ds(eb + HC, HC)], enbB, semi)

        def mk_grp(idx_ref, half):
            def grp(j, carry2):
                for u in range(8):
                    o = half * HC + j * 128 + u * 16
                    idx_ref[pl.ds(j * 128 + u * 16, 16)] = (
                        sidb[pl.ds(o, 16)] + trkb[pl.ds(o, 16)] * S_PAD)
                return carry2
            return grp

        lax.fori_loop(0, HC // 128, mk_grp(idxa, 0), 0)
        dl1.wait()
        dl2.wait()
        dA = pltpu.async_copy(enbA, tbl.at[idxa], sems, add=True)
        lax.fori_loop(0, HC // 128, mk_grp(idxb2, 1), 0)
        dB = pltpu.async_copy(enbB, tbl.at[idxb2], sems, add=True)

        @pl.when(i + 1 < n)
        def _prefetch():
            eb2 = (w + (i + 1) * NW) * CH
            pltpu.async_copy(sid_hbm.at[pl.ds(eb2, CH)], sidb, semi)
            pltpu.async_copy(trk_hbm.at[pl.ds(eb2, CH)], trkb, semi)

        dA.wait()
        dB.wait()
        return carry

    lax.fori_loop(0, n, chunk, 0)
    plsc.subcore_barrier()
    pltpu.sync_copy(tbl.at[pl.ds(s * ZSL, ZSL)],
                    part_hbm.at[pl.ds(c * TBL + s * ZSL, ZSL)])


_seg_sums = pl.kernel(
    _seg_sums_body,
    out_type=jax.ShapeDtypeStruct((NC * TBL,), jnp.float32),
    mesh=_mesh,
    scratch_types=[
        pltpu.VMEM_SHARED((TBL,), jnp.float32),
        pltpu.VMEM((CH,), jnp.int32),
        pltpu.VMEM((CH,), jnp.int32),
        pltpu.VMEM((HC,), jnp.float32),
        pltpu.VMEM((HC,), jnp.float32),
        pltpu.VMEM((HC,), jnp.int32),
        pltpu.VMEM((HC,), jnp.int32),
        pltpu.VMEM((ZSL,), jnp.float32),
        pltpu.SemaphoreType.DMA,
        pltpu.SemaphoreType.DMA,
    ],
)


def _pair_gather_loop(sid_hbm, ta, tb, oa, ob,
                      sid0, sid1, p0, q0, p1, q1,
                      semg, semw0, semw1, sems0, sems1, w):
    n = _num_chunks_for(w)
    m = n // 2
    rem = n - 2 * m
    pltpu.sync_copy(sid_hbm.at[pl.ds(w * CH, CH)], sid0)
    pltpu.sync_copy(sid_hbm.at[pl.ds((w + NW) * CH, CH)], sid1)

    sets = ((sid0, p0, q0, semw0, sems0), (sid1, p1, q1, semw1, sems1))

    def body(j, carry):
        for t, (sb, pbuf, qbuf, semw, sems) in enumerate(sets):
            i = 2 * j + t
            eb = (w + i * NW) * CH

            @pl.when(j > 0)
            def _drain(sb=sb, pbuf=pbuf, qbuf=qbuf, semw=semw, sems=sems, eb=eb):
                pltpu.make_async_copy(pbuf, oa.at[pl.ds(eb, CH)], semw).wait()
                pltpu.make_async_copy(qbuf, ob.at[pl.ds(eb, CH)], semw).wait()
                pltpu.make_async_copy(sid_hbm.at[pl.ds(eb, CH)], sb, sems).wait()

            g1 = pltpu.async_copy(ta.at[sb], pbuf, semg)
            g2 = pltpu.async_copy(tb.at[sb], qbuf, semg)
            g1.wait()
            g2.wait()

            @pl.when(i + 2 < n)
            def _prefetch(sb=sb, sems=sems, i=i):
                eb2 = (w + (i + 2) * NW) * CH
                pltpu.async_copy(sid_hbm.at[pl.ds(eb2, CH)], sb, sems)

            pltpu.async_copy(pbuf, oa.at[pl.ds(eb, CH)], semw)
            pltpu.async_copy(qbuf, ob.at[pl.ds(eb, CH)], semw)
        return carry

    lax.fori_loop(0, m, body, 0)

    @pl.when(rem == 1)
    def _epilogue():
        i = n - 1
        eb = (w + i * NW) * CH
        pltpu.make_async_copy(p0, oa.at[pl.ds(eb, CH)], semw0).wait()
        pltpu.make_async_copy(q0, ob.at[pl.ds(eb, CH)], semw0).wait()
        pltpu.make_async_copy(sid_hbm.at[pl.ds(eb, CH)], sid0, sems0).wait()
        g1 = pltpu.async_copy(ta.at[sid0], p0, semg)
        g2 = pltpu.async_copy(tb.at[sid0], q0, semg)
        g1.wait()
        g2.wait()
        pltpu.sync_copy(p0, oa.at[pl.ds(eb, CH)])
        pltpu.sync_copy(q0, ob.at[pl.ds(eb, CH)])

    @pl.when(rem == 0)
    def _drain_w0():
        pltpu.make_async_copy(p0, oa.at[pl.ds(0, CH)], semw0).wait()
        pltpu.make_async_copy(q0, ob.at[pl.ds(0, CH)], semw0).wait()

    pltpu.make_async_copy(p1, oa.at[pl.ds(0, CH)], semw1).wait()
    pltpu.make_async_copy(q1, ob.at[pl.ds(0, CH)], semw1).wait()


def _track_body(part_hbm, corr_hbm, aih_hbm, ait_hbm, sid_hbm,
                tblh_hbm, otraw, otcor,
                t_traw, t_tcor, pa0, pb0, pa1, pb1, ia0, ia1, cg0, cg1,
                vraw0, vcor0, vraw1, vcor1,
                sid0, sid1, p0, q0, p1, q1,
                semi, semg, semw0, semw1, sems0, sems1):
    s = lax.axis_index("s")
    c = lax.axis_index("c")
    w = _worker_id()
    base = s * SSL

    g = ((pa0, pb0, ia0, cg0, vraw0, vcor0, aih_hbm),
         (pa1, pb1, ia1, cg1, vraw1, vcor1, ait_hbm))
    loads = []
    for gi, (pa, pb, ia, cg, vraw, vcor, alpha_hbm) in enumerate(g):
        off = gi * S_PAD
        loads.append((
            pltpu.async_copy(part_hbm.at[pl.ds(off + base, SSL)], pa, semi),
            pltpu.async_copy(part_hbm.at[pl.ds(TBL + off + base, SSL)], pb, semi),
            pltpu.async_copy(alpha_hbm.at[pl.ds(base, SSL)], cg, semi),
        ))
    for dtrip in loads:
        for d in dtrip:
            d.wait()
    gath = []
    for gi, (pa, pb, ia, cg, vraw, vcor, alpha_hbm) in enumerate(g):

        def cvt(k, carry, ia=ia, cg=cg):
            o = k * 16
            ia[pl.ds(o, 16)] = cg[pl.ds(o, 16)].astype(jnp.int32)
            return carry

        lax.fori_loop(0, SSL // 16, cvt, 0)
        gath.append(pltpu.async_copy(corr_hbm.at[ia], cg, semg))
    gath[0].wait()
    gath[1].wait()
    for gi, (pa, pb, ia, cg, vraw, vcor, alpha_hbm) in enumerate(g):

        def comb(k, carry, pa=pa, pb=pb, cg=cg, vraw=vraw, vcor=vcor):
            o = k * 16
            v = pa[pl.ds(o, 16)] + pb[pl.ds(o, 16)]
            vraw[pl.ds(o, 16)] = v
            vcor[pl.ds(o, 16)] = v * cg[pl.ds(o, 16)]
            return carry

        lax.fori_loop(0, SSL // 16, comb, 0)
        if gi == 0:
            @pl.when(c == 0)
            def _dump():
                pltpu.async_copy(vraw0, tblh_hbm.at[pl.ds(base, SSL)], semw0)
                pltpu.async_copy(vcor0, tblh_hbm.at[pl.ds(S_PAD + base, SSL)], semw0)
        else:
            pltpu.sync_copy(vraw1, t_traw.at[pl.ds(base, SSL)])
            pltpu.sync_copy(vcor1, t_tcor.at[pl.ds(base, SSL)])

    @pl.when(c == 0)
    def _drain_dump():
        pltpu.make_async_copy(vraw0, tblh_hbm.at[pl.ds(base, SSL)], semw0).wait()
        pltpu.make_async_copy(vcor0, tblh_hbm.at[pl.ds(S_PAD + base, SSL)], semw0).wait()

    plsc.subcore_barrier()
    _pair_gather_loop(sid_hbm, t_traw, t_tcor, otraw, otcor,
                      sid0, sid1, p0, q0, p1, q1,
                      semg, semw0, semw1, sems0, sems1, w)


_track_kernel = pl.kernel(
    _track_body,
    out_type=[jax.ShapeDtypeStruct((2 * S_PAD,), jnp.float32),
              jax.ShapeDtypeStruct((NP,), jnp.float32),
              jax.ShapeDtypeStruct((NP,), jnp.float32)],
    mesh=_mesh,
    scratch_types=[
        pltpu.VMEM_SHARED((S_PAD,), jnp.float32),
        pltpu.VMEM_SHARED((S_PAD,), jnp.float32),
        pltpu.VMEM((SSL,), jnp.float32),
        pltpu.VMEM((SSL,), jnp.float32),
        pltpu.VMEM((SSL,), jnp.float32),
        pltpu.VMEM((SSL,), jnp.float32),
        pltpu.VMEM((SSL,), jnp.int32),
        pltpu.VMEM((SSL,), jnp.int32),
        pltpu.VMEM((SSL,), jnp.float32),
        pltpu.VMEM((SSL,), jnp.float32),
        pltpu.VMEM((SSL,), jnp.float32),
        pltpu.VMEM((SSL,), jnp.float32),
        pltpu.VMEM((SSL,), jnp.float32),
        pltpu.VMEM((SSL,), jnp.float32),
        pltpu.VMEM((CH,), jnp.int32),
        pltpu.VMEM((CH,), jnp.int32),
        pltpu.VMEM((CH,), jnp.float32),
        pltpu.VMEM((CH,), jnp.float32),
        pltpu.VMEM((CH,), jnp.float32),
        pltpu.VMEM((CH,), jnp.float32),
        pltpu.SemaphoreType.DMA,
        pltpu.SemaphoreType.DMA,
        pltpu.SemaphoreType.DMA,
        pltpu.SemaphoreType.DMA,
        pltpu.SemaphoreType.DMA,
        pltpu.SemaphoreType.DMA,
    ],
)


def _hit_body(tblh_hbm, sid_hbm, ohraw, ohcor,
              t_hraw, t_hcor, sid0, sid1, p0, q0, p1, q1,
              semi, semg, semw0, semw1, sems0, sems1):
    s = lax.axis_index("s")
    w = _worker_id()
    base = s * SSL
    d1 = pltpu.async_copy(tblh_hbm.at[pl.ds(base, SSL)],
                          t_hraw.at[pl.ds(base, SSL)], semi)
    d2 = pltpu.async_copy(tblh_hbm.at[pl.ds(S_PAD + base, SSL)],
                          t_hcor.at[pl.ds(base, SSL)], semi)
    d1.wait()
    d2.wait()
    plsc.subcore_barrier()
    _pair_gather_loop(sid_hbm, t_hraw, t_hcor, ohraw, ohcor,
                      sid0, sid1, p0, q0, p1, q1,
                      semg, semw0, semw1, sems0, sems1, w)


_hit_kernel = pl.kernel(
    _hit_body,
    out_type=[jax.ShapeDtypeStruct((NP,), jnp.float32)] * 2,
    mesh=_mesh,
    scratch_types=[
        pltpu.VMEM_SHARED((S_PAD,), jnp.float32),
        pltpu.VMEM_SHARED((S_PAD,), jnp.float32),
        pltpu.VMEM((CH,), jnp.int32),
        pltpu.VMEM((CH,), jnp.int32),
        pltpu.VMEM((CH,), jnp.float32),
        pltpu.VMEM((CH,), jnp.float32),
        pltpu.VMEM((CH,), jnp.float32),
        pltpu.VMEM((CH,), jnp.float32),
        pltpu.SemaphoreType.DMA,
        pltpu.SemaphoreType.DMA,
        pltpu.SemaphoreType.DMA,
        pltpu.SemaphoreType.DMA,
        pltpu.SemaphoreType.DMA,
        pltpu.SemaphoreType.DMA,
    ],
)


def _squeeze_pad(x):
    return jnp.pad(x, ((0, NP - N), (0, 0))).reshape(NP)


def kernel(pred_sid, pred_corr_factor, rechit_energy, no_noise_idx,
           pred_beta, is_track, alpha_idx_tracks, alpha_idx_hits):
    del no_noise_idx, pred_beta
    sid1d = _squeeze_pad(pred_sid)
    trk1d = _squeeze_pad(is_track.astype(jnp.int32))
    en1d = _squeeze_pad(rechit_energy)
    corr1d = _squeeze_pad(pred_corr_factor)
    zpad = jnp.zeros((S_PAD - S,), jnp.float32)
    aih1d = jnp.concatenate([alpha_idx_hits.astype(jnp.float32), zpad])
    ait1d = jnp.concatenate([alpha_idx_tracks.astype(jnp.float32), zpad])

    part = _seg_sums(sid1d, trk1d, en1d)
    tblh, otraw, otcor = _track_kernel(part, corr1d, aih1d, ait1d, sid1d)
    ohraw, ohcor = _hit_kernel(tblh, sid1d)

    def rs(x):
        return lax.slice(x.reshape(NP, 1), (0, 0), (N, 1))

    return (rs(otraw), rs(otcor), rs(ohraw), rs(ohcor))


# speedup vs baseline: 181.5022x; 1.1126x over previous
import jax
import jax.numpy as jnp
from jax import lax
from jax.experimental import pallas as pl
from jax.experimental.pallas import tpu as pltpu
from jax.experimental.pallas import tpu_sc as plsc

N = 1_600_000
NP = 1_600_512
S = 100_000
NC = 2
NS = 16
NW = NC * NS

S_PAD = 102_400
TBL = 2 * S_PAD
ZSL = TBL // NS
SSL = S_PAD // NS
CH = 3072
NCHUNKS = NP // CH

_mesh = plsc.VectorSubcoreMesh(
    core_axis_name="c", subcore_axis_name="s", num_cores=NC, num_subcores=NS
)


def _worker_id():
    return lax.axis_index("s") * NC + lax.axis_index("c")


def _num_chunks_for(w):
    return (NCHUNKS - w + NW - 1) // NW


HC = CH // 2


def _seg_sums_body(sid_hbm, trk_hbm, en_hbm, part_hbm,
                   tbl, sidb, trkb, enbA, enbB, idxa, idxb2, zbuf, semi, sems):
    c = lax.axis_index("c")
    s = lax.axis_index("s")
    w = _worker_id()
    n = _num_chunks_for(w)

    def zvec(i, carry):
        zbuf[pl.ds(i * 16, 16)] = jnp.zeros((16,), jnp.float32)
        return carry

    lax.fori_loop(0, ZSL // 16, zvec, 0)
    pltpu.sync_copy(zbuf, tbl.at[pl.ds(s * ZSL, ZSL)])
    plsc.subcore_barrier()

    pltpu.sync_copy(sid_hbm.at[pl.ds(w * CH, CH)], sidb)
    pltpu.sync_copy(trk_hbm.at[pl.ds(w * CH, CH)], trkb)

    def chunk(i, carry):
        eb = (w + i * NW) * CH

        @pl.when(i > 0)
        def _drain_prefetch():
            pltpu.make_async_copy(sid_hbm.at[pl.ds(eb, CH)], sidb, semi).wait()
            pltpu.make_async_copy(trk_hbm.at[pl.ds(eb, CH)], trkb, semi).wait()

        dl1 = pltpu.async_copy(en_hbm.at[pl.ds(eb, HC)], enbA, semi)
        dl2 = pltpu.async_copy(en_hbm.at[pl.ds(eb + HC, HC)], enbB, semi)

        def mk_grp(idx_ref, half):
            def grp(j, carry2):
                for u in range(8):
                    o = half * HC + j * 128 + u * 16
                    idx_ref[pl.ds(j * 128 + u * 16, 16)] = (
                        sidb[pl.ds(o, 16)] + trkb[pl.ds(o, 16)] * S_PAD)
                return carry2
            return grp

        lax.fori_loop(0, HC // 128, mk_grp(idxa, 0), 0)
        dl1.wait()
        dl2.wait()
        dA = pltpu.async_copy(enbA, tbl.at[idxa], sems, add=True)
        lax.fori_loop(0, HC // 128, mk_grp(idxb2, 1), 0)
        dB = pltpu.async_copy(enbB, tbl.at[idxb2], sems, add=True)

        @pl.when(i + 1 < n)
        def _prefetch():
            eb2 = (w + (i + 1) * NW) * CH
            pltpu.async_copy(sid_hbm.at[pl.ds(eb2, CH)], sidb, semi)
            pltpu.async_copy(trk_hbm.at[pl.ds(eb2, CH)], trkb, semi)

        dA.wait()
        dB.wait()
        return carry

    lax.fori_loop(0, n, chunk, 0)
    plsc.subcore_barrier()
    pltpu.sync_copy(tbl.at[pl.ds(s * ZSL, ZSL)],
                    part_hbm.at[pl.ds(c * TBL + s * ZSL, ZSL)])


_seg_sums = pl.kernel(
    _seg_sums_body,
    out_type=jax.ShapeDtypeStruct((NC * TBL,), jnp.float32),
    mesh=_mesh,
    scratch_types=[
        pltpu.VMEM_SHARED((TBL,), jnp.float32),
        pltpu.VMEM((CH,), jnp.int32),
        pltpu.VMEM((CH,), jnp.int32),
        pltpu.VMEM((HC,), jnp.float32),
        pltpu.VMEM((HC,), jnp.float32),
        pltpu.VMEM((HC,), jnp.int32),
        pltpu.VMEM((HC,), jnp.int32),
        pltpu.VMEM((ZSL,), jnp.float32),
        pltpu.SemaphoreType.DMA,
        pltpu.SemaphoreType.DMA,
    ],
)


def _pair_gather_loop(sid_hbm, ta, tb, oa, ob,
                      sid0, sid1, p0, q0, p1, q1,
                      semg, semw0, semw1, sems0, sems1, w):
    n = _num_chunks_for(w)
    m = n // 2
    rem = n - 2 * m
    pltpu.sync_copy(sid_hbm.at[pl.ds(w * CH, CH)], sid0)
    pltpu.sync_copy(sid_hbm.at[pl.ds((w + NW) * CH, CH)], sid1)

    sets = ((sid0, p0, q0, semw0, sems0), (sid1, p1, q1, semw1, sems1))

    def body(j, carry):
        for t, (sb, pbuf, qbuf, semw, sems) in enumerate(sets):
            i = 2 * j + t
            eb = (w + i * NW) * CH

            @pl.when(j > 0)
            def _drain(sb=sb, pbuf=pbuf, qbuf=qbuf, semw=semw, sems=sems, eb=eb):
                pltpu.make_async_copy(pbuf, oa.at[pl.ds(eb, CH)], semw).wait()
                pltpu.make_async_copy(qbuf, ob.at[pl.ds(eb, CH)], semw).wait()
                pltpu.make_async_copy(sid_hbm.at[pl.ds(eb, CH)], sb, sems).wait()

            g1 = pltpu.async_copy(ta.at[sb], pbuf, semg)
            g2 = pltpu.async_copy(tb.at[sb], qbuf, semg)
            g1.wait()
            g2.wait()

            @pl.when(i + 2 < n)
            def _prefetch(sb=sb, sems=sems, i=i):
                eb2 = (w + (i + 2) * NW) * CH
                pltpu.async_copy(sid_hbm.at[pl.ds(eb2, CH)], sb, sems)

            pltpu.async_copy(pbuf, oa.at[pl.ds(eb, CH)], semw)
            pltpu.async_copy(qbuf, ob.at[pl.ds(eb, CH)], semw)
        return carry

    lax.fori_loop(0, m, body, 0)

    @pl.when(rem == 1)
    def _epilogue():
        i = n - 1
        eb = (w + i * NW) * CH
        pltpu.make_async_copy(p0, oa.at[pl.ds(eb, CH)], semw0).wait()
        pltpu.make_async_copy(q0, ob.at[pl.ds(eb, CH)], semw0).wait()
        pltpu.make_async_copy(sid_hbm.at[pl.ds(eb, CH)], sid0, sems0).wait()
        g1 = pltpu.async_copy(ta.at[sid0], p0, semg)
        g2 = pltpu.async_copy(tb.at[sid0], q0, semg)
        g1.wait()
        g2.wait()
        pltpu.sync_copy(p0, oa.at[pl.ds(eb, CH)])
        pltpu.sync_copy(q0, ob.at[pl.ds(eb, CH)])

    @pl.when(rem == 0)
    def _drain_w0():
        pltpu.make_async_copy(p0, oa.at[pl.ds(0, CH)], semw0).wait()
        pltpu.make_async_copy(q0, ob.at[pl.ds(0, CH)], semw0).wait()

    pltpu.make_async_copy(p1, oa.at[pl.ds(0, CH)], semw1).wait()
    pltpu.make_async_copy(q1, ob.at[pl.ds(0, CH)], semw1).wait()


HSL = S_PAD // NW


def _tables_body(part_hbm, corr_hbm, aih_hbm, ait_hbm, tbl4_hbm,
                 pa0, pb0, pa1, pb1, ia0, ia1, cg0, cg1,
                 vr0, vc0, vr1, vc1, semi, semg, semw):
    w = _worker_id()
    base = w * HSL

    g = ((pa0, pb0, ia0, cg0, vr0, vc0, aih_hbm, 2),
         (pa1, pb1, ia1, cg1, vr1, vc1, ait_hbm, 0))
    loads = []
    for pa, pb, ia, cg, vr, vc, alpha_hbm, cbase in g:
        off = (0 if cbase == 2 else S_PAD)
        loads.extend((
            pltpu.async_copy(part_hbm.at[pl.ds(off + base, HSL)], pa, semi),
            pltpu.async_copy(part_hbm.at[pl.ds(TBL + off + base, HSL)], pb, semi),
            pltpu.async_copy(alpha_hbm.at[pl.ds(base, HSL)], cg, semi),
        ))
    for d in loads:
        d.wait()
    gath = []
    for pa, pb, ia, cg, vr, vc, alpha_hbm, cbase in g:
        def cvt(k, carry, ia=ia, cg=cg):
            o = k * 16
            ia[pl.ds(o, 16)] = cg[pl.ds(o, 16)].astype(jnp.int32)
            return carry

        lax.fori_loop(0, HSL // 16, cvt, 0)
        gath.append(pltpu.async_copy(corr_hbm.at[ia], cg, semg))
    gath[0].wait()
    gath[1].wait()
    writes = []
    for pa, pb, ia, cg, vr, vc, alpha_hbm, cbase in g:
        def comb(k, carry, pa=pa, pb=pb, cg=cg, vr=vr, vc=vc):
            o = k * 16
            v = pa[pl.ds(o, 16)] + pb[pl.ds(o, 16)]
            vr[pl.ds(o, 16)] = v
            vc[pl.ds(o, 16)] = v * cg[pl.ds(o, 16)]
            return carry

        lax.fori_loop(0, HSL // 16, comb, 0)
        writes.append(pltpu.async_copy(
            vr, tbl4_hbm.at[pl.ds(cbase * S_PAD + base, HSL)], semw))
        writes.append(pltpu.async_copy(
            vc, tbl4_hbm.at[pl.ds((cbase + 1) * S_PAD + base, HSL)], semw))
    for d in writes:
        d.wait()


_tables_kernel = pl.kernel(
    _tables_body,
    out_type=jax.ShapeDtypeStruct((4 * S_PAD,), jnp.float32),
    mesh=_mesh,
    scratch_types=(
        [pltpu.VMEM((HSL,), jnp.float32)] * 4
        + [pltpu.VMEM((HSL,), jnp.int32)] * 2
        + [pltpu.VMEM((HSL,), jnp.float32)] * 6
        + [pltpu.SemaphoreType.DMA] * 3
    ),
)


def _make_pair_kernel(off_raw, off_cor):
    def body(tbl4_hbm, sid_hbm, oraw, ocor,
             t_raw, t_cor, sid0, sid1, p0, q0, p1, q1,
             semi, semg, semw0, semw1, sems0, sems1):
        s = lax.axis_index("s")
        w = _worker_id()
        base = s * SSL
        d1 = pltpu.async_copy(tbl4_hbm.at[pl.ds(off_raw * S_PAD + base, SSL)],
                              t_raw.at[pl.ds(base, SSL)], semi)
        d2 = pltpu.async_copy(tbl4_hbm.at[pl.ds(off_cor * S_PAD + base, SSL)],
                              t_cor.at[pl.ds(base, SSL)], semi)
        d1.wait()
        d2.wait()
        plsc.subcore_barrier()
        _pair_gather_loop(sid_hbm, t_raw, t_cor, oraw, ocor,
                          sid0, sid1, p0, q0, p1, q1,
                          semg, semw0, semw1, sems0, sems1, w)

    return pl.kernel(
        body,
        out_type=[jax.ShapeDtypeStruct((NP,), jnp.float32)] * 2,
        mesh=_mesh,
        scratch_types=[
            pltpu.VMEM_SHARED((S_PAD,), jnp.float32),
            pltpu.VMEM_SHARED((S_PAD,), jnp.float32),
            pltpu.VMEM((CH,), jnp.int32),
            pltpu.VMEM((CH,), jnp.int32),
            pltpu.VMEM((CH,), jnp.float32),
            pltpu.VMEM((CH,), jnp.float32),
            pltpu.VMEM((CH,), jnp.float32),
            pltpu.VMEM((CH,), jnp.float32),
            pltpu.SemaphoreType.DMA,
            pltpu.SemaphoreType.DMA,
            pltpu.SemaphoreType.DMA,
            pltpu.SemaphoreType.DMA,
            pltpu.SemaphoreType.DMA,
            pltpu.SemaphoreType.DMA,
        ],
    )


_track_pair = _make_pair_kernel(0, 1)
_hit_pair = _make_pair_kernel(2, 3)


def _squeeze_pad(x):
    return jnp.pad(x, ((0, NP - N), (0, 0))).reshape(NP)


def kernel(pred_sid, pred_corr_factor, rechit_energy, no_noise_idx,
           pred_beta, is_track, alpha_idx_tracks, alpha_idx_hits):
    del no_noise_idx, pred_beta
    sid1d = _squeeze_pad(pred_sid)
    trk1d = _squeeze_pad(is_track.astype(jnp.int32))
    en1d = _squeeze_pad(rechit_energy)
    corr1d = _squeeze_pad(pred_corr_factor)
    zpad = jnp.zeros((S_PAD - S,), jnp.float32)
    aih1d = jnp.concatenate([alpha_idx_hits.astype(jnp.float32), zpad])
    ait1d = jnp.concatenate([alpha_idx_tracks.astype(jnp.float32), zpad])

    part = _seg_sums(sid1d, trk1d, en1d)
    tbl4 = _tables_kernel(part, corr1d, aih1d, ait1d)
    otraw, otcor = _track_pair(tbl4, sid1d)
    ohraw, ohcor = _hit_pair(tbl4, sid1d)

    def rs(x):
        return lax.slice(x.reshape(NP, 1), (0, 0), (N, 1))

    return (rs(otraw), rs(otcor), rs(ohraw), rs(ohcor))


# speedup vs baseline: 200.7115x; 1.1058x over previous
import jax
import jax.numpy as jnp
from jax import lax
from jax.experimental import pallas as pl
from jax.experimental.pallas import tpu as pltpu
from jax.experimental.pallas import tpu_sc as plsc

N = 1_600_000
NP = 1_600_512
S = 100_000
NC = 2
NS = 16
NW = NC * NS

S_PAD = 102_400
TBL = 2 * S_PAD
ZSL = TBL // NS
SSL = S_PAD // NS
CH = 3072
NCHUNKS = NP // CH

_mesh = plsc.VectorSubcoreMesh(
    core_axis_name="c", subcore_axis_name="s", num_cores=NC, num_subcores=NS
)


def _worker_id():
    return lax.axis_index("s") * NC + lax.axis_index("c")


def _num_chunks_for(w):
    return (NCHUNKS - w + NW - 1) // NW


HC = CH // 2


def _seg_sums_body(sid_hbm, trk_hbm, en_hbm, part_hbm,
                   tbl, sidb, trkb, enbA, enbB, idxa, idxb2, zbuf, semi, sems):
    c = lax.axis_index("c")
    s = lax.axis_index("s")
    w = _worker_id()
    n = _num_chunks_for(w)

    def zvec(i, carry):
        zbuf[pl.ds(i * 16, 16)] = jnp.zeros((16,), jnp.float32)
        return carry

    lax.fori_loop(0, ZSL // 16, zvec, 0)
    pltpu.sync_copy(zbuf, tbl.at[pl.ds(s * ZSL, ZSL)])
    plsc.subcore_barrier()

    pltpu.sync_copy(sid_hbm.at[pl.ds(w * CH, CH)], sidb)
    pltpu.sync_copy(trk_hbm.at[pl.ds(w * CH, CH)], trkb)

    def chunk(i, carry):
        eb = (w + i * NW) * CH

        @pl.when(i > 0)
        def _drain_prefetch():
            pltpu.make_async_copy(sid_hbm.at[pl.ds(eb, CH)], sidb, semi).wait()
            pltpu.make_async_copy(trk_hbm.at[pl.ds(eb, CH)], trkb, semi).wait()

        dl1 = pltpu.async_copy(en_hbm.at[pl.ds(eb, HC)], enbA, semi)
        dl2 = pltpu.async_copy(en_hbm.at[pl.ds(eb + HC, HC)], enbB, semi)

        def mk_grp(idx_ref, half):
            def grp(j, carry2):
                for u in range(8):
                    o = half * HC + j * 128 + u * 16
                    idx_ref[pl.ds(j * 128 + u * 16, 16)] = (
                        sidb[pl.ds(o, 16)] + trkb[pl.ds(o, 16)] * S_PAD)
                return carry2
            return grp

        lax.fori_loop(0, HC // 128, mk_grp(idxa, 0), 0)
        dl1.wait()
        dl2.wait()
        dA = pltpu.async_copy(enbA, tbl.at[idxa], sems, add=True)
        lax.fori_loop(0, HC // 128, mk_grp(idxb2, 1), 0)
        dB = pltpu.async_copy(enbB, tbl.at[idxb2], sems, add=True)

        @pl.when(i + 1 < n)
        def _prefetch():
            eb2 = (w + (i + 1) * NW) * CH
            pltpu.async_copy(sid_hbm.at[pl.ds(eb2, CH)], sidb, semi)
            pltpu.async_copy(trk_hbm.at[pl.ds(eb2, CH)], trkb, semi)

        dA.wait()
        dB.wait()
        return carry

    lax.fori_loop(0, n, chunk, 0)
    plsc.subcore_barrier()
    pltpu.sync_copy(tbl.at[pl.ds(s * ZSL, ZSL)],
                    part_hbm.at[pl.ds(c * TBL + s * ZSL, ZSL)])


_seg_sums = pl.kernel(
    _seg_sums_body,
    out_type=jax.ShapeDtypeStruct((NC * TBL,), jnp.float32),
    mesh=_mesh,
    scratch_types=[
        pltpu.VMEM_SHARED((TBL,), jnp.float32),
        pltpu.VMEM((CH,), jnp.int32),
        pltpu.VMEM((CH,), jnp.int32),
        pltpu.VMEM((HC,), jnp.float32),
        pltpu.VMEM((HC,), jnp.float32),
        pltpu.VMEM((HC,), jnp.int32),
        pltpu.VMEM((HC,), jnp.int32),
        pltpu.VMEM((ZSL,), jnp.float32),
        pltpu.SemaphoreType.DMA,
        pltpu.SemaphoreType.DMA,
    ],
)


def _single_gather_loop(sid_hbm, ta, oa,
                        sid0, sid1, p0, p1,
                        semg, semw0, semw1, sems0, sems1, w):
    n = _num_chunks_for(w)
    m = n // 2
    rem = n - 2 * m
    pltpu.sync_copy(sid_hbm.at[pl.ds(w * CH, CH)], sid0)
    pltpu.sync_copy(sid_hbm.at[pl.ds((w + NW) * CH, CH)], sid1)

    sets = ((sid0, p0, semw0, sems0), (sid1, p1, semw1, sems1))

    def body(j, carry):
        for t, (sb, pbuf, semw, sems) in enumerate(sets):
            i = 2 * j + t
            eb = (w + i * NW) * CH

            @pl.when(j > 0)
            def _drain(sb=sb, pbuf=pbuf, semw=semw, sems=sems, eb=eb):
                pltpu.make_async_copy(pbuf, oa.at[pl.ds(eb, CH)], semw).wait()
                pltpu.make_async_copy(sid_hbm.at[pl.ds(eb, CH)], sb, sems).wait()

            pltpu.async_copy(ta.at[sb], pbuf, semg).wait()

            @pl.when(i + 2 < n)
            def _prefetch(sb=sb, sems=sems, i=i):
                eb2 = (w + (i + 2) * NW) * CH
                pltpu.async_copy(sid_hbm.at[pl.ds(eb2, CH)], sb, sems)

            pltpu.async_copy(pbuf, oa.at[pl.ds(eb, CH)], semw)
        return carry

    lax.fori_loop(0, m, body, 0)

    @pl.when(rem == 1)
    def _epilogue():
        i = n - 1
        eb = (w + i * NW) * CH
        pltpu.make_async_copy(p0, oa.at[pl.ds(eb, CH)], semw0).wait()
        pltpu.make_async_copy(sid_hbm.at[pl.ds(eb, CH)], sid0, sems0).wait()
        pltpu.async_copy(ta.at[sid0], p0, semg).wait()
        pltpu.sync_copy(p0, oa.at[pl.ds(eb, CH)])

    @pl.when(rem == 0)
    def _drain_w0():
        pltpu.make_async_copy(p0, oa.at[pl.ds(0, CH)], semw0).wait()

    pltpu.make_async_copy(p1, oa.at[pl.ds(0, CH)], semw1).wait()


HSL = S_PAD // NW


def _tables_body(part_hbm, corr_hbm, aih_hbm, ait_hbm, tbl4_hbm,
                 pa0, pb0, pa1, pb1, ia0, ia1, cg0, cg1,
                 vr0, vc0, vr1, vc1, semi, semg, semw):
    w = _worker_id()
    base = w * HSL

    g = ((pa0, pb0, ia0, cg0, vr0, vc0, aih_hbm, 2),
         (pa1, pb1, ia1, cg1, vr1, vc1, ait_hbm, 0))
    loads = []
    for pa, pb, ia, cg, vr, vc, alpha_hbm, cbase in g:
        off = (0 if cbase == 2 else S_PAD)
        loads.extend((
            pltpu.async_copy(part_hbm.at[pl.ds(off + base, HSL)], pa, semi),
            pltpu.async_copy(part_hbm.at[pl.ds(TBL + off + base, HSL)], pb, semi),
            pltpu.async_copy(alpha_hbm.at[pl.ds(base, HSL)], cg, semi),
        ))
    for d in loads:
        d.wait()
    gath = []
    for pa, pb, ia, cg, vr, vc, alpha_hbm, cbase in g:
        def cvt(k, carry, ia=ia, cg=cg):
            o = k * 16
            ia[pl.ds(o, 16)] = cg[pl.ds(o, 16)].astype(jnp.int32)
            return carry

        lax.fori_loop(0, HSL // 16, cvt, 0)
        gath.append(pltpu.async_copy(corr_hbm.at[ia], cg, semg))
    gath[0].wait()
    gath[1].wait()
    writes = []
    for pa, pb, ia, cg, vr, vc, alpha_hbm, cbase in g:
        def comb(k, carry, pa=pa, pb=pb, cg=cg, vr=vr, vc=vc):
            o = k * 16
            v = pa[pl.ds(o, 16)] + pb[pl.ds(o, 16)]
            vr[pl.ds(o, 16)] = v
            vc[pl.ds(o, 16)] = v * cg[pl.ds(o, 16)]
            return carry

        lax.fori_loop(0, HSL // 16, comb, 0)
        writes.append(pltpu.async_copy(
            vr, tbl4_hbm.at[pl.ds(cbase * S_PAD + base, HSL)], semw))
        writes.append(pltpu.async_copy(
            vc, tbl4_hbm.at[pl.ds((cbase + 1) * S_PAD + base, HSL)], semw))
    for d in writes:
        d.wait()


_tables_kernel = pl.kernel(
    _tables_body,
    out_type=jax.ShapeDtypeStruct((4 * S_PAD,), jnp.float32),
    mesh=_mesh,
    scratch_types=(
        [pltpu.VMEM((HSL,), jnp.float32)] * 4
        + [pltpu.VMEM((HSL,), jnp.int32)] * 2
        + [pltpu.VMEM((HSL,), jnp.float32)] * 6
        + [pltpu.SemaphoreType.DMA] * 3
    ),
)


def _make_out_kernel(off):
    def body(tbl4_hbm, sid_hbm, out,
             t_tab, sid0, sid1, p0, p1,
             semi, semg, semw0, semw1, sems0, sems1):
        s = lax.axis_index("s")
        w = _worker_id()
        base = s * SSL
        pltpu.async_copy(tbl4_hbm.at[pl.ds(off * S_PAD + base, SSL)],
                         t_tab.at[pl.ds(base, SSL)], semi).wait()
        plsc.subcore_barrier()
        _single_gather_loop(sid_hbm, t_tab, out,
                            sid0, sid1, p0, p1,
                            semg, semw0, semw1, sems0, sems1, w)

    return pl.kernel(
        body,
        out_type=jax.ShapeDtypeStruct((NP,), jnp.float32),
        mesh=_mesh,
        scratch_types=[
            pltpu.VMEM_SHARED((S_PAD,), jnp.float32),
            pltpu.VMEM((CH,), jnp.int32),
            pltpu.VMEM((CH,), jnp.int32),
            pltpu.VMEM((CH,), jnp.float32),
            pltpu.VMEM((CH,), jnp.float32),
            pltpu.SemaphoreType.DMA,
            pltpu.SemaphoreType.DMA,
            pltpu.SemaphoreType.DMA,
            pltpu.SemaphoreType.DMA,
            pltpu.SemaphoreType.DMA,
            pltpu.SemaphoreType.DMA,
        ],
    )


_out_kernels = tuple(_make_out_kernel(off) for off in range(4))


def _squeeze_pad(x):
    return jnp.pad(x, ((0, NP - N), (0, 0))).reshape(NP)


def kernel(pred_sid, pred_corr_factor, rechit_energy, no_noise_idx,
           pred_beta, is_track, alpha_idx_tracks, alpha_idx_hits):
    del no_noise_idx, pred_beta
    sid1d = _squeeze_pad(pred_sid)
    trk1d = _squeeze_pad(is_track.astype(jnp.int32))
    en1d = _squeeze_pad(rechit_energy)
    corr1d = _squeeze_pad(pred_corr_factor)
    zpad = jnp.zeros((S_PAD - S,), jnp.float32)
    aih1d = jnp.concatenate([alpha_idx_hits.astype(jnp.float32), zpad])
    ait1d = jnp.concatenate([alpha_idx_tracks.astype(jnp.float32), zpad])

    part = _seg_sums(sid1d, trk1d, en1d)
    tbl4 = _tables_kernel(part, corr1d, aih1d, ait1d)
    otraw = _out_kernels[0](tbl4, sid1d)
    otcor = _out_kernels[1](tbl4, sid1d)
    ohraw = _out_kernels[2](tbl4, sid1d)
    ohcor = _out_kernels[3](tbl4, sid1d)

    def rs(x):
        return lax.slice(x.reshape(NP, 1), (0, 0), (N, 1))

    return (rs(otraw), rs(otcor), rs(ohraw), rs(ohcor))
